# Initial kernel scaffold; baseline (speedup 1.0000x reference)
import jax
import jax.numpy as jnp
from jax.experimental import pallas as pl


def kernel(e_emb, b_emb, s_emb, b_curvature, s_curvature, We, be, Wb, bb, Ws, bs, edge_index):
    raise NotImplementedError("write your pallas kernel here")


# speedup vs baseline: 2.1616x; 2.1616x over previous
import functools

import jax
import jax.numpy as jnp
from jax import lax
from jax.experimental import pallas as pl
from jax.experimental.pallas import tpu as pltpu
from jax.experimental.pallas import tpu_sc as plsc

_D = 128
_NC = 2
_NS = 16
_NW = _NC * _NS
_CHUNK = 128
_NB = 6
_DB = 64


def _leaky(x):
    return jnp.where(x >= 0, x, 0.2 * x)


def _rownorm(x):
    return jnp.sqrt(jnp.sum(x * x, axis=-1, keepdims=True))


def _logmap0(y, c):
    sc = jnp.sqrt(c)
    n = jnp.maximum(_rownorm(y), 1e-10)
    z = sc * n
    atanh = 0.5 * jnp.log((1.0 + z) / (1.0 - z))
    return (2.0 / sc) * atanh * y / n


def _expmap0(v, c):
    sc = jnp.sqrt(c)
    n = jnp.maximum(_rownorm(v), 1e-10)
    return jnp.tanh(sc * n / 2.0) * v / (sc * n)


def _normalize(x, eps):
    return x / jnp.maximum(_rownorm(x), eps)


def _matT(x, w):
    return lax.dot_general(x, w, (((1,), (1,)), ((), ())),
                           precision=lax.Precision.HIGHEST,
                           preferred_element_type=jnp.float32)


def _front(e, b, s, we, bev, wb, bbv, ws, bsv, bc, scc):
    he = _matT(e, we) + bev
    tangent = _logmap0(b, bc)
    hb = _matT(tangent, wb) + bbv
    ns = _normalize(s, 1e-12)
    hs = _normalize(_matT(ns, ws) + bsv, 1e-12)
    return he, hb, hs


def _store_blocked(out_ref, he, hb, hs):
    for i, h in enumerate((he, hb, hs)):
        out_ref[2 * i] = h[:, :_DB]
        out_ref[2 * i + 1] = h[:, _DB:]


def _post(p_ref, cnt, bc):
    rinv = 1.0 / jnp.maximum(cnt, 1.0)
    def mean(i):
        lo = (p_ref[0, 2 * i] + p_ref[1, 2 * i]) * rinv
        hi = (p_ref[0, 2 * i + 1] + p_ref[1, 2 * i + 1]) * rinv
        return jnp.concatenate([lo, hi], axis=-1)
    return _leaky(mean(0)), _expmap0(mean(1), bc), _normalize(mean(2), 1e-12)


def _tc_front_body(e_ref, b_ref, s_ref, we_ref, be_ref, wb_ref, bb_ref,
                   ws_ref, bs_ref, cb_ref, cs_ref, out_ref):
    bc = cb_ref[0, 0]
    scc = cs_ref[0, 0]
    he, hb, hs = _front(e_ref[...], b_ref[...], s_ref[...], we_ref[...],
                        be_ref[...], wb_ref[...], bb_ref[...], ws_ref[...],
                        bs_ref[...], bc, scc)
    _store_blocked(out_ref, he, hb, hs)


def _tc_mid_body(p_ref, we_ref, be_ref, wb_ref, bb_ref, ws_ref,
                 bs_ref, cb_ref, cs_ref, out_ref):
    bc = cb_ref[0, 0]
    scc = cs_ref[0, 0]
    cnt = p_ref[0, _NB, :, 0:1] + p_ref[1, _NB, :, 0:1]
    e1, b1, s1 = _post(p_ref, cnt, bc)
    he, hb, hs = _front(e1, b1, s1, we_ref[...], be_ref[...], wb_ref[...],
                        bb_ref[...], ws_ref[...], bs_ref[...], bc, scc)
    _store_blocked(out_ref, he, hb, hs)


def _tc_tail_body(p_ref, c_ref, cb_ref, cs_ref, e_ref, b_ref, s_ref):
    bc = cb_ref[0, 0]
    cnt = c_ref[0, :, 0:1] + c_ref[1, :, 0:1]
    e1, b1, s1 = _post(p_ref, cnt, bc)
    e_ref[...] = e1
    b_ref[...] = b1
    s_ref[...] = s1


def _make_tc_front(npad, blk=512):
    row = pl.BlockSpec((blk, _D), lambda i: (i, 0))
    wsp = pl.BlockSpec((_D, _D), lambda i: (0, 0))
    bsp = pl.BlockSpec((1, _D), lambda i: (0, 0))
    ssp = pl.BlockSpec(memory_space=pltpu.SMEM)
    return pl.pallas_call(
        _tc_front_body,
        grid=(npad // blk,),
        in_specs=[row, row, row, wsp, bsp, wsp, bsp, wsp, bsp, ssp, ssp],
        out_specs=pl.BlockSpec((_NB, blk, _DB), lambda i: (0, i, 0)),
        out_shape=jax.ShapeDtypeStruct((_NB, npad, _DB), jnp.float32),
    )


def _make_tc_mid(npad, blk=512):
    psp = pl.BlockSpec((_NC, _NB + 1, blk, _DB), lambda i: (0, 0, i, 0))
    wsp = pl.BlockSpec((_D, _D), lambda i: (0, 0))
    bsp = pl.BlockSpec((1, _D), lambda i: (0, 0))
    ssp = pl.BlockSpec(memory_space=pltpu.SMEM)
    return pl.pallas_call(
        _tc_mid_body,
        grid=(npad // blk,),
        in_specs=[psp, wsp, bsp, wsp, bsp, wsp, bsp, ssp, ssp],
        out_specs=pl.BlockSpec((_NB, blk, _DB), lambda i: (0, i, 0)),
        out_shape=jax.ShapeDtypeStruct((_NB, npad, _DB), jnp.float32),
    )


def _make_tc_tail(npad, blk=512):
    psp = pl.BlockSpec((_NC, _NB, blk, _DB), lambda i: (0, 0, i, 0))
    csp = pl.BlockSpec((_NC, blk, _DB), lambda i: (0, i, 0))
    ssp = pl.BlockSpec(memory_space=pltpu.SMEM)
    row = pl.BlockSpec((blk, _D), lambda i: (i, 0))
    shp = jax.ShapeDtypeStruct((npad, _D), jnp.float32)
    return pl.pallas_call(
        _tc_tail_body,
        grid=(npad // blk,),
        in_specs=[psp, csp, ssp, ssp],
        out_specs=(row, row, row),
        out_shape=(shp, shp, shp),
    )


def _make_sc_segsum(npad, nchunk, with_count):
    rpt = npad // _NS
    nz = rpt // _CHUNK
    nb = _NB + 1 if with_count else _NB
    mesh = plsc.VectorSubcoreMesh(core_axis_name="c", subcore_axis_name="s",
                                  num_cores=_NC, num_subcores=_NS)

    @functools.partial(
        pl.kernel,
        out_type=jax.ShapeDtypeStruct((_NC, nb, npad, _DB), jnp.float32),
        mesh=mesh,
        scratch_types=[
            pltpu.VMEM((_NB, nchunk, _CHUNK), jnp.int32),
            pltpu.VMEM((nchunk, _CHUNK), jnp.int32),
            pltpu.VMEM((_CHUNK, _DB), jnp.float32),
            pltpu.VMEM((_CHUNK, _DB), jnp.float32),
            pltpu.VMEM_SHARED((npad, _DB), jnp.float32),
            pltpu.SemaphoreType.DMA,
        ],
        compiler_params=pltpu.CompilerParams(use_tc_tiling_on_sc=False),
    )
    def segsum(h_hbm, src6_hbm, dst_hbm, ones_hbm, zeros_hbm,
               out_hbm, src6_v, dst_v, buf, obuf, acc, sem):
        ci = lax.axis_index("c")
        si = lax.axis_index("s")
        w = ci * _NS + si
        pltpu.sync_copy(src6_hbm.at[w], src6_v)
        pltpu.sync_copy(dst_hbm.at[w], dst_v)
        pltpu.sync_copy(zeros_hbm, obuf)
        r0 = si * rpt

        def zero_acc():
            for k in range(nz):
                pltpu.sync_copy(obuf, acc.at[pl.ds(r0 + k * _CHUNK, _CHUNK)])

        zero_acc()
        plsc.subcore_barrier()

        for b in range(nb):
            if b < _NB:
                def body(j, carry, _b=b):
                    pltpu.async_copy(h_hbm.at[src6_v.at[_b, j]], buf,
                                     sem).wait()
                    pltpu.sync_copy(buf, acc.at[dst_v.at[j]], add=True)
                    return carry
            else:
                def body(j, carry):
                    pltpu.sync_copy(obuf, acc.at[dst_v.at[j]], add=True)
                    return carry
            if b == _NB:
                pltpu.sync_copy(ones_hbm, obuf)
            lax.fori_loop(0, nchunk, body, 0)
            plsc.subcore_barrier()
            pltpu.sync_copy(acc.at[pl.ds(r0, rpt)],
                            out_hbm.at[ci, b, pl.ds(r0, rpt)])
            if b < nb - 1:
                if b == _NB - 1 and with_count:
                    pltpu.sync_copy(zeros_hbm, buf)
                    for k in range(nz):
                        pltpu.sync_copy(
                            buf, acc.at[pl.ds(r0 + k * _CHUNK, _CHUNK)])
                else:
                    zero_acc()
                plsc.subcore_barrier()

    return segsum


def kernel(e_emb, b_emb, s_emb, b_curvature, s_curvature, We, be, Wb, bb,
           Ws, bs, edge_index):
    n, d = e_emb.shape
    assert d == _D
    e = edge_index.shape[1]
    npad = -(-n // 640) * 640
    nchunk = -(-e // (_NW * _CHUNK))
    ep = _NW * nchunk * _CHUNK

    bc = jnp.asarray(b_curvature, jnp.float32).reshape(1, 1)
    scc = jnp.asarray(s_curvature, jnp.float32).reshape(1, 1)

    src = edge_index[0]
    dst = edge_index[1]
    srcp = jnp.concatenate([src, jnp.zeros((ep - e,), jnp.int32)])
    dstp = jnp.concatenate([dst, jnp.full((ep - e,), n, jnp.int32)])
    src6 = (srcp[None, :] +
            (jnp.arange(_NB, dtype=jnp.int32) * npad)[:, None])
    src6 = src6.reshape(_NB, _NW, nchunk, _CHUNK).transpose(1, 0, 2, 3)
    dst4 = dstp.reshape(_NW, nchunk, _CHUNK)
    zeros = jnp.zeros((_CHUNK, _DB), jnp.float32)
    ones = jnp.ones((_CHUNK, _DB), jnp.float32)

    pr = npad - n
    ez = jnp.pad(e_emb, ((0, pr), (0, 0)))
    bz = jnp.pad(b_emb, ((0, pr), (0, 0)))
    sz = jnp.pad(s_emb, ((0, pr), (0, 0)))

    segsum_c = _make_sc_segsum(npad, nchunk, True)
    segsum_n = _make_sc_segsum(npad, nchunk, False)
    tc_front = _make_tc_front(npad)
    tc_mid = _make_tc_mid(npad)
    tc_tail = _make_tc_tail(npad)

    h0 = tc_front(ez, bz, sz,
                  We[0], be[0].reshape(1, _D), Wb[0], bb[0].reshape(1, _D),
                  Ws[0], bs[0].reshape(1, _D), bc, scc)
    p0 = segsum_c(h0.reshape(_NB * npad, _DB), src6, dst4, ones, zeros)
    h1 = tc_mid(p0,
                We[1], be[1].reshape(1, _D), Wb[1], bb[1].reshape(1, _D),
                Ws[1], bs[1].reshape(1, _D), bc, scc)
    p1 = segsum_n(h1.reshape(_NB * npad, _DB), src6, dst4, ones, zeros)
    e2, b2, s2 = tc_tail(p1, p0[:, _NB], bc, scc)
    return e2[:n], b2[:n], s2[:n]


# speedup vs baseline: 2.5895x; 1.1979x over previous
import functools

import jax
import jax.numpy as jnp
from jax import lax
from jax.experimental import pallas as pl
from jax.experimental.pallas import tpu as pltpu
from jax.experimental.pallas import tpu_sc as plsc

_D = 128
_NC = 2
_NS = 16
_NW = _NC * _NS
_CHUNK = 64
_NB = 6
_DB = 64


def _leaky(x):
    return jnp.where(x >= 0, x, 0.2 * x)


def _rownorm(x):
    return jnp.sqrt(jnp.sum(x * x, axis=-1, keepdims=True))


def _logmap0(y, c):
    sc = jnp.sqrt(c)
    n = jnp.maximum(_rownorm(y), 1e-10)
    z = sc * n
    atanh = 0.5 * jnp.log((1.0 + z) / (1.0 - z))
    return (2.0 / sc) * atanh * y / n


def _expmap0(v, c):
    sc = jnp.sqrt(c)
    n = jnp.maximum(_rownorm(v), 1e-10)
    return jnp.tanh(sc * n / 2.0) * v / (sc * n)


def _normalize(x, eps):
    return x / jnp.maximum(_rownorm(x), eps)


def _matT(x, w):
    return lax.dot_general(x, w, (((1,), (1,)), ((), ())),
                           precision=lax.Precision.HIGHEST,
                           preferred_element_type=jnp.float32)


def _front(e, b, s, we, bev, wb, bbv, ws, bsv, bc, scc):
    he = _matT(e, we) + bev
    tangent = _logmap0(b, bc)
    hb = _matT(tangent, wb) + bbv
    ns = _normalize(s, 1e-12)
    hs = _normalize(_matT(ns, ws) + bsv, 1e-12)
    return he, hb, hs


def _store_blocked(out_ref, he, hb, hs):
    full = jnp.concatenate([he, hb, hs], axis=-1)
    for i in range(_NB):
        out_ref[i] = full[:, i * _DB:(i + 1) * _DB]


def _post(p_ref, cnt, bc):
    rinv = 1.0 / jnp.maximum(cnt, 1.0)
    full = jnp.concatenate(
        [p_ref[0, i] + p_ref[1, i] for i in range(_NB)], axis=-1) * rinv
    me, mb, ms = full[:, :_D], full[:, _D:2 * _D], full[:, 2 * _D:]
    return _leaky(me), _expmap0(mb, bc), _normalize(ms, 1e-12)


def _tc_front_body(e_ref, b_ref, s_ref, we_ref, be_ref, wb_ref, bb_ref,
                   ws_ref, bs_ref, cb_ref, cs_ref, out_ref):
    bc = cb_ref[0, 0]
    scc = cs_ref[0, 0]
    he, hb, hs = _front(e_ref[...], b_ref[...], s_ref[...], we_ref[...],
                        be_ref[...], wb_ref[...], bb_ref[...], ws_ref[...],
                        bs_ref[...], bc, scc)
    _store_blocked(out_ref, he, hb, hs)


def _tc_mid_body(p_ref, c_ref, we_ref, be_ref, wb_ref, bb_ref, ws_ref,
                 bs_ref, cb_ref, cs_ref, out_ref):
    bc = cb_ref[0, 0]
    scc = cs_ref[0, 0]
    cnt = c_ref[0, :, 0:1] + c_ref[1, :, 0:1]
    e1, b1, s1 = _post(p_ref, cnt, bc)
    he, hb, hs = _front(e1, b1, s1, we_ref[...], be_ref[...], wb_ref[...],
                        bb_ref[...], ws_ref[...], bs_ref[...], bc, scc)
    _store_blocked(out_ref, he, hb, hs)


def _tc_tail_body(p_ref, c_ref, cb_ref, cs_ref, e_ref, b_ref, s_ref):
    bc = cb_ref[0, 0]
    cnt = c_ref[0, :, 0:1] + c_ref[1, :, 0:1]
    e1, b1, s1 = _post(p_ref, cnt, bc)
    e_ref[...] = e1
    b_ref[...] = b1
    s_ref[...] = s1


def _make_tc_front(npad, blk=512):
    row = pl.BlockSpec((blk, _D), lambda i: (i, 0))
    wsp = pl.BlockSpec((_D, _D), lambda i: (0, 0))
    bsp = pl.BlockSpec((1, _D), lambda i: (0, 0))
    ssp = pl.BlockSpec(memory_space=pltpu.SMEM)
    return pl.pallas_call(
        _tc_front_body,
        grid=(npad // blk,),
        in_specs=[row, row, row, wsp, bsp, wsp, bsp, wsp, bsp, ssp, ssp],
        out_specs=pl.BlockSpec((_NB, blk, _DB), lambda i: (0, i, 0)),
        out_shape=jax.ShapeDtypeStruct((_NB, npad, _DB), jnp.float32),
    )


def _make_tc_mid(npad, blk=512):
    psp = pl.BlockSpec((_NC, _NB, blk, _DB), lambda i: (0, 0, i, 0))
    csp = pl.BlockSpec((_NC, blk, 16), lambda i: (0, i, 0))
    wsp = pl.BlockSpec((_D, _D), lambda i: (0, 0))
    bsp = pl.BlockSpec((1, _D), lambda i: (0, 0))
    ssp = pl.BlockSpec(memory_space=pltpu.SMEM)
    return pl.pallas_call(
        _tc_mid_body,
        grid=(npad // blk,),
        in_specs=[psp, csp, wsp, bsp, wsp, bsp, wsp, bsp, ssp, ssp],
        out_specs=pl.BlockSpec((_NB, blk, _DB), lambda i: (0, i, 0)),
        out_shape=jax.ShapeDtypeStruct((_NB, npad, _DB), jnp.float32),
    )


def _make_tc_tail(npad, blk=512):
    psp = pl.BlockSpec((_NC, _NB, blk, _DB), lambda i: (0, 0, i, 0))
    csp = pl.BlockSpec((_NC, blk, 16), lambda i: (0, i, 0))
    ssp = pl.BlockSpec(memory_space=pltpu.SMEM)
    row = pl.BlockSpec((blk, _D), lambda i: (i, 0))
    shp = jax.ShapeDtypeStruct((npad, _D), jnp.float32)
    return pl.pallas_call(
        _tc_tail_body,
        grid=(npad // blk,),
        in_specs=[psp, csp, ssp, ssp],
        out_specs=(row, row, row),
        out_shape=(shp, shp, shp),
    )


def _make_sc_segsum(npad, nchunk, with_count):
    rpt = npad // _NS
    nz = rpt // _CHUNK
    nb = _NB
    mesh = plsc.VectorSubcoreMesh(core_axis_name="c", subcore_axis_name="s",
                                  num_cores=_NC, num_subcores=_NS)

    total = _NB * nchunk

    @functools.partial(
        pl.kernel,
        out_type=jax.ShapeDtypeStruct((_NC, nb, npad, _DB), jnp.float32),
        mesh=mesh,
        scratch_types=[
            pltpu.VMEM((total, _CHUNK), jnp.int32),
            pltpu.VMEM((nchunk, _CHUNK), jnp.int32),
            pltpu.VMEM((2, _CHUNK, _DB), jnp.float32),
            pltpu.VMEM((_CHUNK, _DB), jnp.float32),
            pltpu.VMEM_SHARED((npad, _DB), jnp.float32),
            pltpu.SemaphoreType.DMA((2,)),
        ],
        compiler_params=pltpu.CompilerParams(use_tc_tiling_on_sc=False),
    )
    def segsum(h_hbm, src6_hbm, dst_hbm, ones_hbm, zeros_hbm,
               out_hbm, src6_v, dst_v, buf, obuf, acc, sem):
        del ones_hbm
        ci = lax.axis_index("c")
        si = lax.axis_index("s")
        w = ci * _NS + si
        pltpu.sync_copy(src6_hbm.at[w], src6_v)
        pltpu.sync_copy(dst_hbm.at[w], dst_v)
        pltpu.sync_copy(zeros_hbm, obuf)
        r0 = si * rpt

        def zero_acc():
            for k in range(nz):
                pltpu.sync_copy(obuf, acc.at[pl.ds(r0 + k * _CHUNK, _CHUNK)])

        zero_acc()
        plsc.subcore_barrier()

        def prime(j, c2):
            pltpu.async_copy(h_hbm.at[src6_v.at[j]], buf.at[j], sem.at[j])
            return c2
        lax.fori_loop(0, 2, prime, 0)

        def body(q, c2):
            par = lax.rem(q, 2)
            j = lax.rem(q, nchunk)
            b = lax.div(q, nchunk)
            nxt = jnp.minimum(q + 2, total - 1)
            pltpu.make_async_copy(h_hbm.at[src6_v.at[q]], buf.at[par],
                                  sem.at[par]).wait()
            pltpu.sync_copy(buf.at[par], acc.at[dst_v.at[j]], add=True)
            pltpu.async_copy(h_hbm.at[src6_v.at[nxt]], buf.at[par],
                             sem.at[par])

            @pl.when(j == nchunk - 1)
            def _flush():
                plsc.subcore_barrier()
                pltpu.sync_copy(acc.at[pl.ds(r0, rpt)],
                                out_hbm.at[ci, b, pl.ds(r0, rpt)])
                zero_acc()
                plsc.subcore_barrier()
            return c2
        lax.fori_loop(0, total, body, 0)

        def drain(j, c2):
            pltpu.make_async_copy(h_hbm.at[src6_v.at[0]], buf.at[j],
                                  sem.at[j]).wait()
            return c2
        lax.fori_loop(0, 2, drain, 0)

    return segsum


def _make_sc_count(npad, nchunk):
    rpt = npad // _NS
    nz = rpt // _CHUNK
    mesh = plsc.VectorSubcoreMesh(core_axis_name="c", subcore_axis_name="s",
                                  num_cores=_NC, num_subcores=_NS)

    @functools.partial(
        pl.kernel,
        out_type=jax.ShapeDtypeStruct((_NC, npad, 16), jnp.float32),
        mesh=mesh,
        scratch_types=[
            pltpu.VMEM((nchunk, _CHUNK), jnp.int32),
            pltpu.VMEM((_CHUNK, 16), jnp.float32),
            pltpu.VMEM_SHARED((npad, 16), jnp.float32),
        ],
        compiler_params=pltpu.CompilerParams(use_tc_tiling_on_sc=False),
    )
    def count(dst_hbm, ones_hbm, zeros_hbm, cnt_hbm, dst_v, obuf, acc16):
        ci = lax.axis_index("c")
        si = lax.axis_index("s")
        w = ci * _NS + si
        pltpu.sync_copy(dst_hbm.at[w], dst_v)
        pltpu.sync_copy(zeros_hbm, obuf)
        r0 = si * rpt
        for k in range(nz):
            pltpu.sync_copy(obuf, acc16.at[pl.ds(r0 + k * _CHUNK, _CHUNK)])
        pltpu.sync_copy(ones_hbm, obuf)
        plsc.subcore_barrier()

        def body(j, carry):
            pltpu.sync_copy(obuf, acc16.at[dst_v.at[j]], add=True)
            return carry
        lax.fori_loop(0, nchunk, body, 0)
        plsc.subcore_barrier()
        pltpu.sync_copy(acc16.at[pl.ds(r0, rpt)],
                        cnt_hbm.at[ci, pl.ds(r0, rpt)])

    return count


def kernel(e_emb, b_emb, s_emb, b_curvature, s_curvature, We, be, Wb, bb,
           Ws, bs, edge_index):
    n, d = e_emb.shape
    assert d == _D
    e = edge_index.shape[1]
    npad = -(-n // 640) * 640
    nchunk = 2 * -(-e // (_NW * _CHUNK * 2))
    ep = _NW * nchunk * _CHUNK

    bc = jnp.asarray(b_curvature, jnp.float32).reshape(1, 1)
    scc = jnp.asarray(s_curvature, jnp.float32).reshape(1, 1)

    src = edge_index[0]
    dst = edge_index[1]
    srcp = jnp.concatenate([src, jnp.zeros((ep - e,), jnp.int32)])
    dstp = jnp.concatenate([dst, jnp.full((ep - e,), n, jnp.int32)])
    src6 = (srcp[None, :] +
            (jnp.arange(_NB, dtype=jnp.int32) * npad)[:, None])
    src6 = src6.reshape(_NB, _NW, nchunk, _CHUNK).transpose(1, 0, 2, 3)
    src6 = src6.reshape(_NW, _NB * nchunk, _CHUNK)
    dst4 = dstp.reshape(_NW, nchunk, _CHUNK)
    zeros = jnp.zeros((_CHUNK, _DB), jnp.float32)
    ones = jnp.ones((_CHUNK, _DB), jnp.float32)
    zeros16 = jnp.zeros((_CHUNK, 16), jnp.float32)
    ones16 = jnp.ones((_CHUNK, 16), jnp.float32)

    pr = npad - n
    ez = jnp.pad(e_emb, ((0, pr), (0, 0)))
    bz = jnp.pad(b_emb, ((0, pr), (0, 0)))
    sz = jnp.pad(s_emb, ((0, pr), (0, 0)))

    segsum = _make_sc_segsum(npad, nchunk, False)
    sc_count = _make_sc_count(npad, nchunk)
    tc_front = _make_tc_front(npad)
    tc_mid = _make_tc_mid(npad)
    tc_tail = _make_tc_tail(npad)

    cnt = sc_count(dst4, ones16, zeros16)
    h0 = tc_front(ez, bz, sz,
                  We[0], be[0].reshape(1, _D), Wb[0], bb[0].reshape(1, _D),
                  Ws[0], bs[0].reshape(1, _D), bc, scc)
    p0 = segsum(h0.reshape(_NB * npad, _DB), src6, dst4, ones, zeros)
    h1 = tc_mid(p0, cnt,
                We[1], be[1].reshape(1, _D), Wb[1], bb[1].reshape(1, _D),
                Ws[1], bs[1].reshape(1, _D), bc, scc)
    p1 = segsum(h1.reshape(_NB * npad, _DB), src6, dst4, ones, zeros)
    e2, b2, s2 = tc_tail(p1, cnt, bc, scc)
    return e2[:n], b2[:n], s2[:n]
